# baseline (speedup 1.0000x reference)
import functools

import jax
import jax.numpy as jnp
from jax import lax
from jax.experimental import pallas as pl
from jax.experimental.pallas import tpu as pltpu
from jax.experimental.pallas import tpu_sc as plsc

B = 512
D = 64
INPUT = 256
HID = 128
M = 100000
K = 5
CHUNK = 10000
NCHUNK = M // CHUNK
DECAY = 0.001
NCLS = 10

SC_NC = 2
SC_NS = 16
SC_NW = SC_NC * SC_NS
IDX_PER_W = (B * K) // SC_NW
OUT_PER_W = B // SC_NW


def _topk_body(x_ref, w1_ref, b1_ref, w2_ref, b2_ref, keys_ref, ages_ref,
               z_ref, w_ref, idx_ref, zn_s, rv_s, rn_s):
    i = pl.program_id(0)

    @pl.when(i == 0)
    def _init():
        h = jnp.maximum(
            jnp.dot(x_ref[...], w1_ref[...],
                    preferred_element_type=jnp.float32) + b1_ref[...][None, :],
            0.0)
        z = jnp.dot(h, w2_ref[...],
                    preferred_element_type=jnp.float32) + b2_ref[...][None, :]
        z_ref[...] = z
        zn_s[...] = z / (jnp.sqrt(jnp.sum(z * z, axis=1, keepdims=True)) + 1e-8)
        rv_s[...] = jnp.full((B, 8), -jnp.inf, jnp.float32)
        rn_s[...] = jnp.zeros((B, 8), jnp.float32)

    gbig = jnp.float32(1 << 20)
    h = CHUNK // 2
    keys = keys_ref[...]
    kn = keys * lax.rsqrt(jnp.sum(keys * keys, axis=1, keepdims=True))
    ages = ages_ref[...]
    zn = zn_s[...]
    a = (lax.dot_general(zn, kn[:h], (((1,), (1,)), ((), ())),
                         preferred_element_type=jnp.float32)
         * jnp.exp(-DECAY * ages[:, 0, :]))
    b = (lax.dot_general(zn, kn[h:], (((1,), (1,)), ((), ())),
                         preferred_element_type=jnp.float32)
         * jnp.exp(-DECAY * ages[:, 1, :]))

    base = i * CHUNK
    neg_a = (gbig - base) - lax.broadcasted_iota(
        jnp.int32, (B, h), 1).astype(jnp.float32)
    neg_b = neg_a - h
    sel_a = a >= b
    p = jnp.where(sel_a, a, b)
    q = jnp.where(sel_a, b, a)
    pcode = jnp.where(sel_a, neg_a, neg_b)
    qcode = jnp.where(sel_a, neg_b, neg_a)
    rv = rv_s[...]
    rn = rn_s[...]
    new_v = []
    new_n = []
    for _ in range(K):
        m = jnp.maximum(jnp.max(p, axis=1, keepdims=True),
                        jnp.max(rv, axis=1, keepdims=True))
        am_c = jnp.max(jnp.where(p == m, pcode, 0.0), axis=1, keepdims=True)
        am_r = jnp.max(jnp.where(rv == m, rn, 0.0), axis=1, keepdims=True)
        g = jnp.maximum(am_c, am_r)
        hit = pcode == g
        p = jnp.where(hit, q, p)
        pcode = jnp.where(hit, qcode, pcode)
        q = jnp.where(hit, -jnp.inf, q)
        rv = jnp.where(rn == g, -jnp.inf, rv)
        new_v.append(m)
        new_n.append(g)
    rv_new = jnp.concatenate(new_v + [jnp.full((B, 3), -jnp.inf, jnp.float32)],
                             axis=1)
    rn_new = jnp.concatenate(new_n + [jnp.zeros((B, 3), jnp.float32)], axis=1)
    rv_s[...] = rv_new
    rn_s[...] = rn_new

    @pl.when(i == NCHUNK - 1)
    def _finish():
        mm = jnp.max(rv_new, axis=1, keepdims=True)
        e = jnp.exp(rv_new - mm)
        w_ref[...] = e / jnp.sum(e, axis=1, keepdims=True)
        idx_ref[...] = (gbig - rn_new).astype(jnp.int32)


def _topk_call(x, w1, b1, w2, b2, keys, ages3):
    return pl.pallas_call(
        _topk_body,
        grid=(NCHUNK,),
        in_specs=[
            pl.BlockSpec((B, INPUT), lambda i: (0, 0)),
            pl.BlockSpec((INPUT, HID), lambda i: (0, 0)),
            pl.BlockSpec((HID,), lambda i: (0,)),
            pl.BlockSpec((HID, D), lambda i: (0, 0)),
            pl.BlockSpec((D,), lambda i: (0,)),
            pl.BlockSpec((CHUNK, D), lambda i: (i, 0)),
            pl.BlockSpec((1, 2, CHUNK // 2), lambda i: (i, 0, 0)),
        ],
        out_specs=[
            pl.BlockSpec((B, D), lambda i: (0, 0)),
            pl.BlockSpec((B, 8), lambda i: (0, 0)),
            pl.BlockSpec((B, 8), lambda i: (0, 0)),
        ],
        out_shape=[
            jax.ShapeDtypeStruct((B, D), jnp.float32),
            jax.ShapeDtypeStruct((B, 8), jnp.float32),
            jax.ShapeDtypeStruct((B, 8), jnp.int32),
        ],
        scratch_shapes=[
            pltpu.VMEM((B, D), jnp.float32),
            pltpu.VMEM((B, 8), jnp.float32),
            pltpu.VMEM((B, 8), jnp.float32),
        ],
        compiler_params=pltpu.CompilerParams(
            dimension_semantics=("arbitrary",)),
    )(x, w1, b1, w2, b2, keys, ages3)


def _sc_gather_nat_body(values_hbm, idx_hbm, w_hbm, out_hbm,
                        idx_v, w_v, rows_v, acc_v, sem):
    wid = lax.axis_index("s") * SC_NC + lax.axis_index("c")
    base = wid * IDX_PER_W
    pltpu.sync_copy(idx_hbm.at[pl.ds(base, IDX_PER_W)], idx_v)
    pltpu.sync_copy(w_hbm.at[pl.ds(base, IDX_PER_W)], w_v)
    pltpu.async_copy(values_hbm.at[idx_v], rows_v, sem).wait()
    for r in range(OUT_PER_W):
        wl = [w_v[K * r + k, :] for k in range(K)]
        for d in range(D // 16):
            acc = wl[0] * rows_v[K * r, pl.ds(d * 16, 16)]
            for k in range(1, K):
                acc = acc + wl[k] * rows_v[K * r + k, pl.ds(d * 16, 16)]
            acc_v[r, pl.ds(d * 16, 16)] = acc
    pltpu.sync_copy(acc_v, out_hbm.at[pl.ds(wid * OUT_PER_W, OUT_PER_W)])


def _make_sc_gather_nat():
    return functools.partial(
        pl.kernel,
        mesh=plsc.VectorSubcoreMesh(core_axis_name="c",
                                    subcore_axis_name="s"),
        out_type=jax.ShapeDtypeStruct((B, D), jnp.float32),
        scratch_types=[
            pltpu.VMEM((IDX_PER_W,), jnp.int32),
            pltpu.VMEM((IDX_PER_W, 16), jnp.float32),
            pltpu.VMEM((IDX_PER_W, D), jnp.float32),
            pltpu.VMEM((OUT_PER_W, D), jnp.float32),
            pltpu.SemaphoreType.DMA,
        ],
        compiler_params=pltpu.CompilerParams(use_tc_tiling_on_sc=False),
    )(_sc_gather_nat_body)


def _sc_gather_body(values2_hbm, idx2_hbm, wlo_hbm, whi_hbm, out_hbm,
                    idx_v, wlo_v, whi_v, rows_v, acc_v, sem):
    wid = lax.axis_index("s") * SC_NC + lax.axis_index("c")
    base = wid * IDX_PER_W
    pltpu.sync_copy(idx2_hbm.at[pl.ds(base, IDX_PER_W)], idx_v)
    pltpu.sync_copy(wlo_hbm.at[pl.ds(base, IDX_PER_W)], wlo_v)
    pltpu.sync_copy(whi_hbm.at[pl.ds(base, IDX_PER_W)], whi_v)
    pltpu.async_copy(values2_hbm.at[idx_v], rows_v, sem).wait()
    for r in range(OUT_PER_W):
        wl = [wlo_v[K * r + k, :] for k in range(K)]
        wh = [whi_v[K * r + k, :] for k in range(K)]
        for d in range(D // 16):
            acc = wl[0] * rows_v[K * r, pl.ds(d * 16, 16)]
            acc = acc + wh[0] * rows_v[K * r, pl.ds(D + d * 16, 16)]
            for k in range(1, K):
                acc = acc + wl[k] * rows_v[K * r + k, pl.ds(d * 16, 16)]
                acc = acc + wh[k] * rows_v[K * r + k, pl.ds(D + d * 16, 16)]
            acc_v[r, pl.ds(d * 16, 16)] = acc
    pltpu.sync_copy(acc_v, out_hbm.at[pl.ds(wid * OUT_PER_W, OUT_PER_W)])


def _make_sc_gather():
    return functools.partial(
        pl.kernel,
        mesh=plsc.VectorSubcoreMesh(core_axis_name="c",
                                    subcore_axis_name="s"),
        out_type=jax.ShapeDtypeStruct((B, D), jnp.float32),
        scratch_types=[
            pltpu.VMEM((IDX_PER_W,), jnp.int32),
            pltpu.VMEM((IDX_PER_W, 16), jnp.float32),
            pltpu.VMEM((IDX_PER_W, 16), jnp.float32),
            pltpu.VMEM((IDX_PER_W, 2 * D), jnp.float32),
            pltpu.VMEM((OUT_PER_W, D), jnp.float32),
            pltpu.SemaphoreType.DMA,
        ],
    )(_sc_gather_body)


def _dec_body(z_ref, r_ref, wd1_ref, bd1_ref, wd2_ref, bd2_ref, out_ref):
    hd = jnp.maximum(
        jnp.dot(z_ref[...], wd1_ref[:D, :],
                preferred_element_type=jnp.float32)
        + jnp.dot(r_ref[...], wd1_ref[D:, :],
                  preferred_element_type=jnp.float32)
        + bd1_ref[...][None, :], 0.0)
    out_ref[...] = jnp.dot(hd, wd2_ref[...],
                           preferred_element_type=jnp.float32) \
        + bd2_ref[...][None, :]


def _dec_call(z, recall, wd1, bd1, wd2, bd2):
    return pl.pallas_call(
        _dec_body,
        out_shape=jax.ShapeDtypeStruct((B, NCLS), jnp.float32),
    )(z, recall, wd1, bd1, wd2, bd2)


def kernel(x, memory_keys, memory_values, memory_ages, W1, b1, W2, b2,
           Wd1, bd1, Wd2, bd2):
    ages3 = memory_ages.reshape(NCHUNK, 2, CHUNK // 2)
    z, w8, idx8 = _topk_call(x, W1, b1, W2, b2, memory_keys, ages3)
    idx_flat = idx8[:, :K].reshape(-1)
    w_b = jnp.broadcast_to(w8[:, :K].reshape(-1)[:, None], (B * K, 16))
    recall = _make_sc_gather_nat()(memory_values, idx_flat, w_b)
    logits = _dec_call(z, recall, Wd1, bd1, Wd2, bd2)
    return (logits, z)


# speedup vs baseline: 1.0612x; 1.0612x over previous
import functools

import jax
import jax.numpy as jnp
from jax import lax
from jax.experimental import pallas as pl
from jax.experimental.pallas import tpu as pltpu
from jax.experimental.pallas import tpu_sc as plsc

B = 512
D = 64
INPUT = 256
HID = 128
M = 100000
K = 5
CHUNK = 10000
NCHUNK = M // CHUNK
DECAY = 0.001
NCLS = 10

SC_NC = 2
SC_NS = 16
SC_NW = SC_NC * SC_NS
IDX_PER_W = (B * K) // SC_NW
OUT_PER_W = B // SC_NW


def _topk_body(x_ref, w1_ref, b1_ref, w2_ref, b2_ref, keys_ref, ages_ref,
               z_ref, w_ref, idx_ref, zn_s, rv_s, rn_s):
    i = pl.program_id(0)

    @pl.when(i == 0)
    def _init():
        h = jnp.maximum(
            jnp.dot(x_ref[...], w1_ref[...],
                    preferred_element_type=jnp.float32) + b1_ref[...][None, :],
            0.0)
        z = jnp.dot(h, w2_ref[...],
                    preferred_element_type=jnp.float32) + b2_ref[...][None, :]
        z_ref[...] = z
        zn_s[...] = z / (jnp.sqrt(jnp.sum(z * z, axis=1, keepdims=True)) + 1e-8)
        rv_s[...] = jnp.full((B, 8), -jnp.inf, jnp.float32)
        rn_s[...] = jnp.zeros((B, 8), jnp.float32)

    gbig = jnp.float32(1 << 20)
    keys = keys_ref[...]
    kn = keys * lax.rsqrt(jnp.sum(keys * keys, axis=1, keepdims=True))
    decay = jnp.exp(-DECAY * ages_ref[...].reshape(1, CHUNK))
    sim = lax.dot_general(zn_s[...], kn, (((1,), (1,)), ((), ())),
                          preferred_element_type=jnp.float32)
    scores = sim * decay

    base = i * CHUNK
    negio = (gbig - base) - lax.broadcasted_iota(
        jnp.int32, (B, CHUNK), 1).astype(jnp.float32)
    rv = rv_s[...]
    rn = rn_s[...]
    new_v = []
    new_n = []
    for _ in range(K):
        m = jnp.maximum(jnp.max(scores, axis=1, keepdims=True),
                        jnp.max(rv, axis=1, keepdims=True))
        am_c = jnp.max(jnp.where(scores == m, negio, 0.0),
                       axis=1, keepdims=True)
        am_r = jnp.max(jnp.where(rv == m, rn, 0.0), axis=1, keepdims=True)
        g = jnp.maximum(am_c, am_r)
        scores = jnp.where(negio == g, -jnp.inf, scores)
        rv = jnp.where(rn == g, -jnp.inf, rv)
        new_v.append(m)
        new_n.append(g)
    rv_new = jnp.concatenate(new_v + [jnp.full((B, 3), -jnp.inf, jnp.float32)],
                             axis=1)
    rn_new = jnp.concatenate(new_n + [jnp.zeros((B, 3), jnp.float32)], axis=1)
    rv_s[...] = rv_new
    rn_s[...] = rn_new

    @pl.when(i == NCHUNK - 1)
    def _finish():
        mm = jnp.max(rv_new, axis=1, keepdims=True)
        e = jnp.exp(rv_new - mm)
        w_ref[...] = e / jnp.sum(e, axis=1, keepdims=True)
        idx_ref[...] = (gbig - rn_new).astype(jnp.int32)


def _topk_call(x, w1, b1, w2, b2, keys, ages3):
    return pl.pallas_call(
        _topk_body,
        grid=(NCHUNK,),
        in_specs=[
            pl.BlockSpec((B, INPUT), lambda i: (0, 0)),
            pl.BlockSpec((INPUT, HID), lambda i: (0, 0)),
            pl.BlockSpec((HID,), lambda i: (0,)),
            pl.BlockSpec((HID, D), lambda i: (0, 0)),
            pl.BlockSpec((D,), lambda i: (0,)),
            pl.BlockSpec((CHUNK, D), lambda i: (i, 0)),
            pl.BlockSpec((1, 1, CHUNK), lambda i: (i, 0, 0)),
        ],
        out_specs=[
            pl.BlockSpec((B, D), lambda i: (0, 0)),
            pl.BlockSpec((B, 8), lambda i: (0, 0)),
            pl.BlockSpec((B, 8), lambda i: (0, 0)),
        ],
        out_shape=[
            jax.ShapeDtypeStruct((B, D), jnp.float32),
            jax.ShapeDtypeStruct((B, 8), jnp.float32),
            jax.ShapeDtypeStruct((B, 8), jnp.int32),
        ],
        scratch_shapes=[
            pltpu.VMEM((B, D), jnp.float32),
            pltpu.VMEM((B, 8), jnp.float32),
            pltpu.VMEM((B, 8), jnp.float32),
        ],
        compiler_params=pltpu.CompilerParams(
            dimension_semantics=("arbitrary",)),
    )(x, w1, b1, w2, b2, keys, ages3)


def _sc_gather_nat_body(values_hbm, idx_hbm, w_hbm, out_hbm,
                        idx_v, w_v, rows_v, acc_v, sem):
    wid = lax.axis_index("s") * SC_NC + lax.axis_index("c")
    base = wid * IDX_PER_W
    pltpu.sync_copy(idx_hbm.at[pl.ds(base, IDX_PER_W)], idx_v)
    pltpu.sync_copy(w_hbm.at[pl.ds(base, IDX_PER_W)], w_v)
    pltpu.async_copy(values_hbm.at[idx_v], rows_v, sem).wait()
    for r in range(OUT_PER_W):
        wl = [w_v[K * r + k, :] for k in range(K)]
        for d in range(D // 16):
            acc = wl[0] * rows_v[K * r, pl.ds(d * 16, 16)]
            for k in range(1, K):
                acc = acc + wl[k] * rows_v[K * r + k, pl.ds(d * 16, 16)]
            acc_v[r, pl.ds(d * 16, 16)] = acc
    pltpu.sync_copy(acc_v, out_hbm.at[pl.ds(wid * OUT_PER_W, OUT_PER_W)])


def _make_sc_gather_nat():
    return functools.partial(
        pl.kernel,
        mesh=plsc.VectorSubcoreMesh(core_axis_name="c",
                                    subcore_axis_name="s"),
        out_type=jax.ShapeDtypeStruct((B, D), jnp.float32),
        scratch_types=[
            pltpu.VMEM((IDX_PER_W,), jnp.int32),
            pltpu.VMEM((IDX_PER_W, 16), jnp.float32),
            pltpu.VMEM((IDX_PER_W, D), jnp.float32),
            pltpu.VMEM((OUT_PER_W, D), jnp.float32),
            pltpu.SemaphoreType.DMA,
        ],
        compiler_params=pltpu.CompilerParams(use_tc_tiling_on_sc=False),
    )(_sc_gather_nat_body)


def _dec_body(z_ref, r_ref, wd1_ref, bd1_ref, wd2_ref, bd2_ref, out_ref):
    hd = jnp.maximum(
        jnp.dot(z_ref[...], wd1_ref[:D, :],
                preferred_element_type=jnp.float32)
        + jnp.dot(r_ref[...], wd1_ref[D:, :],
                  preferred_element_type=jnp.float32)
        + bd1_ref[...][None, :], 0.0)
    out_ref[...] = jnp.dot(hd, wd2_ref[...],
                           preferred_element_type=jnp.float32) \
        + bd2_ref[...][None, :]


def _dec_call(z, recall, wd1, bd1, wd2, bd2):
    return pl.pallas_call(
        _dec_body,
        out_shape=jax.ShapeDtypeStruct((B, NCLS), jnp.float32),
    )(z, recall, wd1, bd1, wd2, bd2)


def kernel(x, memory_keys, memory_values, memory_ages, W1, b1, W2, b2,
           Wd1, bd1, Wd2, bd2):
    ages3 = memory_ages.reshape(NCHUNK, 1, CHUNK)
    z, w8, idx8 = _topk_call(x, W1, b1, W2, b2, memory_keys, ages3)
    idx_flat = idx8[:, :K].reshape(-1)
    w_b = jnp.broadcast_to(w8[:, :K].reshape(-1)[:, None], (B * K, 16))
    recall = _make_sc_gather_nat()(memory_values, idx_flat, w_b)
    logits = _dec_call(z, recall, Wd1, bd1, Wd2, bd2)
    return (logits, z)
